# baseline (speedup 1.0000x reference)
import functools

import jax
import jax.numpy as jnp
from jax import lax
from jax.experimental import pallas as pl
from jax.experimental.pallas import tpu as pltpu, tpu_sc as plsc

B = 16384
D = 64
NC = 2
NS = 16
NW = NC * NS
BPW = B // NW
NCHUNK = BPW // 128


def _sc_kernel(user_table, item_table, uids2d, iids2d, out_hbm,
               idx_u, idx_i, u_rows, i_rows, out_v, sem):
    wid = lax.axis_index("s") * NC + lax.axis_index("c")

    pltpu.sync_copy(uids2d.at[pl.ds(wid * NCHUNK, NCHUNK)], idx_u)
    pltpu.sync_copy(iids2d.at[pl.ds(wid * NCHUNK, NCHUNK)], idx_i)

    copies = []
    for c in range(NCHUNK):
        copies.append(pltpu.async_copy(
            user_table.at[idx_u.at[c]], u_rows.at[pl.ds(c * 128, 128)], sem))
        copies.append(pltpu.async_copy(
            item_table.at[idx_i.at[c]], i_rows.at[pl.ds(c * 128, 128)], sem))
    for cp in copies:
        cp.wait()

    lane = lax.iota(jnp.int32, 16)
    perms = [(lane + sh) & 15 for sh in (8, 4, 2, 1)]

    def group(g, carry):
        acc = jnp.zeros((16,), jnp.float32)
        for t in range(16):
            b = g * 16 + t
            p = u_rows[b, pl.ds(0, 16)] * i_rows[b, pl.ds(0, 16)]
            for j in range(1, D // 16):
                p += u_rows[b, pl.ds(j * 16, 16)] * i_rows[b, pl.ds(j * 16, 16)]
            for perm in perms:
                p = p + p.at[perm].get(mode="promise_in_bounds")
            acc = jnp.where(lane == t, p, acc)
        out_v[pl.ds(g * 16, 16)] = acc
        return carry

    lax.fori_loop(0, BPW // 16, group, 0)

    pltpu.sync_copy(out_v, out_hbm.at[pl.ds(wid * BPW, BPW)])


@jax.jit
def kernel(user_ids, item_ids, user_table, item_table):
    uids2d = user_ids.reshape(NW * NCHUNK, 128)
    iids2d = item_ids.reshape(NW * NCHUNK, 128)
    mesh = plsc.VectorSubcoreMesh(core_axis_name="c", subcore_axis_name="s")
    run = functools.partial(
        pl.kernel, mesh=mesh,
        compiler_params=pltpu.CompilerParams(use_tc_tiling_on_sc=False),
        out_type=jax.ShapeDtypeStruct((B,), jnp.float32),
        scratch_types=[
            pltpu.VMEM((NCHUNK, 128), jnp.int32),
            pltpu.VMEM((NCHUNK, 128), jnp.int32),
            pltpu.VMEM((BPW, D), jnp.float32),
            pltpu.VMEM((BPW, D), jnp.float32),
            pltpu.VMEM((BPW,), jnp.float32),
            pltpu.SemaphoreType.DMA,
        ],
    )(_sc_kernel)
    return run(user_table, item_table, uids2d, iids2d)
